# Initial kernel scaffold; baseline (speedup 1.0000x reference)
import jax
import jax.numpy as jnp
from jax.experimental import pallas as pl


def kernel(inputs, w_one_hot, w_numeric, v_one_hot, v_numeric, b):
    raise NotImplementedError("write your pallas kernel here")


# speedup vs baseline: 1.5265x; 1.5265x over previous
import functools

import jax
import jax.numpy as jnp
from jax import lax
from jax.experimental import pallas as pl
from jax.experimental.pallas import tpu as pltpu
from jax.experimental.pallas import tpu_sc as plsc

NC = 2
NS = 16
NW = NC * NS
L = 16

B = 16384
F = 26
EMB = 16
NUMERIC = 13
V = 1_000_000

ROWS_PER_W = B // NW
CHUNK = 64
NCHUNK = ROWS_PER_W // CHUNK
IDX_PER_CHUNK = CHUNK * F
IDX_TILE = 128
TILES_PER_CHUNK = IDX_PER_CHUNK // IDX_TILE
IDX_ROWS_PER_W = ROWS_PER_W * F // IDX_TILE


def _tree_add(vs):
    while len(vs) > 1:
        nxt = [vs[i] + vs[i + 1] for i in range(0, len(vs) - 1, 2)]
        if len(vs) % 2:
            nxt.append(vs[-1])
        vs = nxt
    return vs[0]


def _sc_fm_stats(idx2, v_tab, w_tab):
    mesh = plsc.VectorSubcoreMesh(
        core_axis_name="c", subcore_axis_name="s",
        num_cores=NC, num_subcores=NS)

    @functools.partial(
        pl.kernel,
        out_type=(
            jax.ShapeDtypeStruct((B, EMB), jnp.float32),
            jax.ShapeDtypeStruct((B, EMB), jnp.float32),
            jax.ShapeDtypeStruct((B,), jnp.float32),
        ),
        mesh=mesh,
        compiler_params=pltpu.CompilerParams(use_tc_tiling_on_sc=False),
        scratch_types=(
            pltpu.VMEM((IDX_ROWS_PER_W, IDX_TILE), jnp.int32),
            pltpu.VMEM((2, IDX_PER_CHUNK, EMB), jnp.float32),
            pltpu.VMEM((2, IDX_PER_CHUNK), jnp.float32),
            pltpu.VMEM((2, CHUNK, EMB), jnp.float32),
            pltpu.VMEM((2, CHUNK, EMB), jnp.float32),
            pltpu.VMEM((2, CHUNK), jnp.float32),
            pltpu.SemaphoreType.DMA,
            pltpu.SemaphoreType.DMA,
        ),
    )
    def sc_kernel(idx_hbm, v_hbm, w_hbm, s_out, q_out, w_out,
                  idx_all, vrows, wvals, s_stage, q_stage, w_stage,
                  gsem, osem):
        wid = lax.axis_index("s") * NC + lax.axis_index("c")
        idx_row0 = wid * IDX_ROWS_PER_W
        pltpu.sync_copy(idx_hbm.at[pl.ds(idx_row0, IDX_ROWS_PER_W)], idx_all)

        def issue_gathers(c):
            slot = c % 2
            descs = []
            for i in range(TILES_PER_CHUNK):
                irow = idx_all.at[c * TILES_PER_CHUNK + i]
                dst_v = vrows.at[slot].at[pl.ds(i * IDX_TILE, IDX_TILE)]
                descs.append(pltpu.async_copy(v_hbm.at[irow], dst_v, gsem))
                dst_w = wvals.at[slot].at[pl.ds(i * IDX_TILE, IDX_TILE)]
                descs.append(pltpu.async_copy(w_hbm.at[irow], dst_w, gsem))
            return descs

        def compute(c):
            slot = c % 2
            vr = vrows.at[slot]
            wv = wvals.at[slot]

            def row_body(r, carry):
                vs = [vr[j * CHUNK + r, :] for j in range(F)]
                s_stage[slot, r, :] = _tree_add(vs)
                q_stage[slot, r, :] = _tree_add([v * v for v in vs])
                return carry

            lax.fori_loop(0, CHUNK, row_body, 0)

            def fo_body(j, accs):
                return tuple(
                    accs[k] + wv[pl.ds(j * CHUNK + k * L, L)]
                    for k in range(CHUNK // L))

            zero = jnp.zeros((L,), jnp.float32)
            accs = lax.fori_loop(0, F, fo_body, (zero,) * (CHUNK // L))
            for k in range(CHUNK // L):
                w_stage[slot, pl.ds(k * L, L)] = accs[k]

        def issue_out(c):
            slot = c % 2
            ob = wid * ROWS_PER_W + c * CHUNK
            return [
                pltpu.async_copy(s_stage.at[slot], s_out.at[pl.ds(ob, CHUNK)], osem),
                pltpu.async_copy(q_stage.at[slot], q_out.at[pl.ds(ob, CHUNK)], osem),
                pltpu.async_copy(w_stage.at[slot], w_out.at[pl.ds(ob, CHUNK)], osem),
            ]

        gd = {0: issue_gathers(0)}
        od = {}
        for c in range(NCHUNK):
            if c + 1 < NCHUNK:
                gd[c + 1] = issue_gathers(c + 1)
            for d in gd.pop(c):
                d.wait()
            if c - 2 in od:
                for d in od.pop(c - 2):
                    d.wait()
            compute(c)
            od[c] = issue_out(c)
        for descs in od.values():
            for d in descs:
                d.wait()

    return sc_kernel(idx2, v_tab, w_tab)


def _tc_combine(num_ref, vn_ref, wn_ref, b_ref, s_ref, q_ref, w_ref, o_ref):
    x = num_ref[...]
    vn = vn_ref[...]
    sn = jnp.dot(x, vn, preferred_element_type=jnp.float32)
    qn = jnp.dot(x * x, vn * vn, preferred_element_type=jnp.float32)
    st = s_ref[...] + sn
    first = w_ref[...] + jnp.sum(x * wn_ref[...], axis=1, keepdims=True)
    second = 0.5 * jnp.sum(st * st - q_ref[...] - qn, axis=1, keepdims=True)
    o_ref[...] = first + second + b_ref[0, 0]


def kernel(inputs, w_one_hot, w_numeric, v_one_hot, v_numeric, b):
    idx = inputs[:, :F].astype(jnp.int32)
    idx2 = (idx.reshape(NW, NCHUNK, CHUNK, F)
               .transpose(0, 1, 3, 2)
               .reshape(B * F // IDX_TILE, IDX_TILE))
    numeric = inputs[:, F:]
    s, q, wsum = _sc_fm_stats(idx2, v_one_hot, w_one_hot.reshape(V))
    out = pl.pallas_call(
        _tc_combine,
        out_shape=jax.ShapeDtypeStruct((B, 1), jnp.float32),
    )(numeric, v_numeric, w_numeric.reshape(1, NUMERIC), b.reshape(1, 1),
      s, q, wsum.reshape(B, 1))
    return out
